# baseline (speedup 1.0000x reference)
import functools

import jax
import jax.numpy as jnp
from jax.experimental import pallas as pl
from jax.experimental.pallas import tpu as pltpu

N_CODES = 1024
DIM = 64
B = 16384
BB = 1024
NB = B // BB


def _vq_kernel(z_ref, cb_ref, zq_ref, enc_ref, idx_ref,
               closs_ref, plex_ref, usage_ref,
               cbm2_sc, cbh_sc, cn_sc, counts_acc, closs_acc):
    i = pl.program_id(0)

    @pl.when(i == 0)
    def _prologue():
        cb0 = cb_ref[...]
        cbm2_sc[...] = -2.0 * cb0
        cbh_sc[...] = cb0.astype(jnp.bfloat16)
        cn_sc[...] = jnp.sum(cb0 * cb0, axis=1, keepdims=True)

    z = z_ref[...]
    zn = jnp.sum(z * z, axis=1, keepdims=True)
    znT = jax.lax.transpose(zn, (1, 0))

    sm2T = jax.lax.dot_general(cbm2_sc[...], z, (((1,), (1,)), ((), ())),
                               preferred_element_type=jnp.float32)
    dT = (znT + cn_sc[...]) + sm2T
    dminT = jnp.min(dT, axis=0, keepdims=True)
    iota0 = jax.lax.broadcasted_iota(jnp.int32, (N_CODES, BB), 0)
    idxT = jnp.min(jnp.where(dT == dminT, iota0, N_CODES), axis=0,
                   keepdims=True)
    idx_col = idxT.reshape(BB, 1)
    iota1 = jax.lax.broadcasted_iota(jnp.int32, (BB, N_CODES), 1)
    hit = iota1 == idx_col
    onehot = hit.astype(jnp.float32)
    zq = jax.lax.dot_general(hit.astype(jnp.bfloat16), cbh_sc[...],
                             (((1,), (0,)), ((), ())),
                             preferred_element_type=jnp.float32)

    zq_ref[...] = zq
    enc_ref[...] = onehot
    idx_ref[...] = idxT.astype(jnp.int32)[None]

    block_counts = jnp.sum(onehot, axis=0, keepdims=True)
    block_closs = jnp.sum(dminT)

    @pl.when(i == 0)
    def _init():
        counts_acc[...] = block_counts
        closs_acc[0, 0] = block_closs

    @pl.when(i > 0)
    def _acc():
        counts_acc[...] += block_counts
        closs_acc[0, 0] += block_closs

    @pl.when(i == NB - 1)
    def _finalize():
        counts = counts_acc[...]
        avg = counts * (1.0 / B)
        plex = jnp.exp(-jnp.sum(avg * jnp.log(avg + 1e-10)))
        usage = jnp.mean((avg > 0.001).astype(jnp.float32))
        closs_ref[...] = jnp.full((1, 1), closs_acc[0, 0] * (1.0 / (B * DIM)),
                                  jnp.float32)
        plex_ref[...] = jnp.full((1, 1), plex, jnp.float32)
        usage_ref[...] = jnp.full((1, 1), usage, jnp.float32)


@functools.partial(jax.jit, static_argnames=())
def kernel(z_e, codebook):
    z = z_e.astype(jnp.float32)
    cb = codebook.astype(jnp.float32)

    out_shapes = (
        jax.ShapeDtypeStruct((B, DIM), jnp.float32),
        jax.ShapeDtypeStruct((B, N_CODES), jnp.float32),
        jax.ShapeDtypeStruct((NB, 1, BB), jnp.int32),
        jax.ShapeDtypeStruct((1, 1), jnp.float32),
        jax.ShapeDtypeStruct((1, 1), jnp.float32),
        jax.ShapeDtypeStruct((1, 1), jnp.float32),
    )
    grid = (NB,)
    in_specs = [
        pl.BlockSpec((BB, DIM), lambda i: (i, 0)),
        pl.BlockSpec((N_CODES, DIM), lambda i: (0, 0)),
    ]
    out_specs = (
        pl.BlockSpec((BB, DIM), lambda i: (i, 0)),
        pl.BlockSpec((BB, N_CODES), lambda i: (i, 0)),
        pl.BlockSpec((1, 1, BB), lambda i: (i, 0, 0)),
        pl.BlockSpec((1, 1), lambda i: (0, 0)),
        pl.BlockSpec((1, 1), lambda i: (0, 0)),
        pl.BlockSpec((1, 1), lambda i: (0, 0)),
    )
    zq, enc, idx3, closs, plex, usage = pl.pallas_call(
        _vq_kernel,
        grid=grid,
        in_specs=in_specs,
        out_specs=out_specs,
        out_shape=out_shapes,
        scratch_shapes=[
            pltpu.VMEM((N_CODES, DIM), jnp.float32),
            pltpu.VMEM((N_CODES, DIM), jnp.bfloat16),
            pltpu.VMEM((N_CODES, 1), jnp.float32),
            pltpu.VMEM((1, N_CODES), jnp.float32),
            pltpu.SMEM((1, 1), jnp.float32),
        ],
    )(z, cb)

    indices = idx3.reshape(B)
    return (zq.astype(z_e.dtype),
            closs.reshape(()),
            plex.reshape(()),
            usage.reshape(()),
            indices,
            enc.astype(z_e.dtype))


# speedup vs baseline: 1.0036x; 1.0036x over previous
import functools

import jax
import jax.numpy as jnp
from jax.experimental import pallas as pl
from jax.experimental.pallas import tpu as pltpu

N_CODES = 1024
DIM = 64
B = 16384
BB = 1024
NB = B // BB


def _vq_kernel(z_ref, cbh_ref, cbm2_ref, zn_ref, cn_ref, zq_ref, enc_ref,
               idx_ref, closs_ref, plex_ref, usage_ref, counts_acc,
               closs_acc):
    i = pl.program_id(0)

    z = z_ref[...]
    cbm2 = cbm2_ref[...]
    zn = zn_ref[...]
    cn = cn_ref[...]

    sm2T = jax.lax.dot_general(cbm2, z, (((1,), (1,)), ((), ())),
                               preferred_element_type=jnp.float32)
    dT = (zn + cn) + sm2T
    dminT = jnp.min(dT, axis=0, keepdims=True)
    iota0 = jax.lax.broadcasted_iota(jnp.int32, (N_CODES, BB), 0)
    idxT = jnp.min(jnp.where(dT == dminT, iota0, N_CODES), axis=0,
                   keepdims=True)
    idx_col = idxT.reshape(BB, 1)
    iota1 = jax.lax.broadcasted_iota(jnp.int32, (BB, N_CODES), 1)
    hit = iota1 == idx_col
    zq = jax.lax.dot_general(hit.astype(jnp.bfloat16), cbh_ref[...],
                             (((1,), (0,)), ((), ())),
                             preferred_element_type=jnp.float32)
    onehot = hit.astype(jnp.float32)

    zq_ref[...] = zq
    enc_ref[...] = onehot
    idx_ref[...] = idxT.astype(jnp.int32)[None]

    block_counts = jnp.sum(onehot, axis=0, keepdims=True)
    block_closs = jnp.sum(dminT)

    @pl.when(i == 0)
    def _init():
        counts_acc[...] = block_counts
        closs_acc[0, 0] = block_closs

    @pl.when(i > 0)
    def _acc():
        counts_acc[...] += block_counts
        closs_acc[0, 0] += block_closs

    @pl.when(i == NB - 1)
    def _finalize():
        counts = counts_acc[...]
        avg = counts * (1.0 / B)
        plex = jnp.exp(-jnp.sum(avg * jnp.log(avg + 1e-10)))
        usage = jnp.mean((avg > 0.001).astype(jnp.float32))
        closs_ref[...] = jnp.full((1, 1), closs_acc[0, 0] * (1.0 / (B * DIM)),
                                  jnp.float32)
        plex_ref[...] = jnp.full((1, 1), plex, jnp.float32)
        usage_ref[...] = jnp.full((1, 1), usage, jnp.float32)


@functools.partial(jax.jit, static_argnames=())
def kernel(z_e, codebook):
    z = z_e.astype(jnp.float32)
    cb = codebook.astype(jnp.float32)
    cbh = cb.astype(jnp.bfloat16)
    cbm2 = -2.0 * cb
    zn = jnp.sum(z * z, axis=1)[None, :]
    cn = jnp.sum(cb * cb, axis=1)[:, None]

    out_shapes = (
        jax.ShapeDtypeStruct((B, DIM), jnp.float32),
        jax.ShapeDtypeStruct((B, N_CODES), jnp.float32),
        jax.ShapeDtypeStruct((NB, 1, BB), jnp.int32),
        jax.ShapeDtypeStruct((1, 1), jnp.float32),
        jax.ShapeDtypeStruct((1, 1), jnp.float32),
        jax.ShapeDtypeStruct((1, 1), jnp.float32),
    )
    grid = (NB,)
    in_specs = [
        pl.BlockSpec((BB, DIM), lambda i: (i, 0)),
        pl.BlockSpec((N_CODES, DIM), lambda i: (0, 0)),
        pl.BlockSpec((N_CODES, DIM), lambda i: (0, 0)),
        pl.BlockSpec((1, BB), lambda i: (0, i)),
        pl.BlockSpec((N_CODES, 1), lambda i: (0, 0)),
    ]
    out_specs = (
        pl.BlockSpec((BB, DIM), lambda i: (i, 0)),
        pl.BlockSpec((BB, N_CODES), lambda i: (i, 0)),
        pl.BlockSpec((1, 1, BB), lambda i: (i, 0, 0)),
        pl.BlockSpec((1, 1), lambda i: (0, 0)),
        pl.BlockSpec((1, 1), lambda i: (0, 0)),
        pl.BlockSpec((1, 1), lambda i: (0, 0)),
    )
    zq, enc, idx3, closs, plex, usage = pl.pallas_call(
        _vq_kernel,
        grid=grid,
        in_specs=in_specs,
        out_specs=out_specs,
        out_shape=out_shapes,
        scratch_shapes=[
            pltpu.VMEM((1, N_CODES), jnp.float32),
            pltpu.SMEM((1, 1), jnp.float32),
        ],
    )(z, cbh, cbm2, zn, cn)

    indices = idx3.reshape(B)
    return (zq, closs.reshape(()), plex.reshape(()), usage.reshape(()),
            indices, enc)


# speedup vs baseline: 1.0140x; 1.0103x over previous
import functools

import jax
import jax.numpy as jnp
from jax.experimental import pallas as pl
from jax.experimental.pallas import tpu as pltpu

N_CODES = 1024
DIM = 64
B = 16384
BB = 1024
NB = B // BB


def _vq_kernel(z_ref, cb_ref, cbm2_ref, zn_ref, cn_ref, zq_ref, enc_ref,
               idx_ref, closs_ref, plex_ref, usage_ref, counts_acc,
               closs_acc):
    i = pl.program_id(0)

    z = z_ref[...]
    cb = cb_ref[...]
    cbm2 = cbm2_ref[...]
    zn = zn_ref[...]
    cn = cn_ref[...]

    sm2T = jax.lax.dot_general(cbm2, z, (((1,), (1,)), ((), ())),
                               preferred_element_type=jnp.float32)
    dT = (zn + cn) + sm2T
    dminT = jnp.min(dT, axis=0, keepdims=True)
    iota0 = jax.lax.broadcasted_iota(jnp.int32, (N_CODES, BB), 0)
    idxT = jnp.min(jnp.where(dT == dminT, iota0, N_CODES), axis=0,
                   keepdims=True)
    idx_col = idxT.reshape(BB, 1)
    iota1 = jax.lax.broadcasted_iota(jnp.int32, (BB, N_CODES), 1)
    onehot = (iota1 == idx_col).astype(jnp.float32)
    zq = jax.lax.dot_general(onehot, cb, (((1,), (0,)), ((), ())),
                             preferred_element_type=jnp.float32)

    zq_ref[...] = zq
    enc_ref[...] = onehot
    idx_ref[...] = idxT.astype(jnp.int32)[None]

    block_counts = jnp.sum(onehot, axis=0, keepdims=True)
    block_closs = jnp.sum(dminT)

    @pl.when(i == 0)
    def _init():
        counts_acc[...] = block_counts
        closs_acc[0, 0] = block_closs

    @pl.when(i > 0)
    def _acc():
        counts_acc[...] += block_counts
        closs_acc[0, 0] += block_closs

    @pl.when(i == NB - 1)
    def _finalize():
        counts = counts_acc[...]
        avg = counts * (1.0 / B)
        plex = jnp.exp(-jnp.sum(avg * jnp.log(avg + 1e-10)))
        usage = jnp.mean((avg > 0.001).astype(jnp.float32))
        closs_ref[...] = jnp.full((1, 1), closs_acc[0, 0] * (1.0 / (B * DIM)),
                                  jnp.float32)
        plex_ref[...] = jnp.full((1, 1), plex, jnp.float32)
        usage_ref[...] = jnp.full((1, 1), usage, jnp.float32)


@functools.partial(jax.jit, static_argnames=())
def kernel(z_e, codebook):
    z = z_e.astype(jnp.float32)
    cb = codebook.astype(jnp.float32)
    cbm2 = -2.0 * cb
    zn = jnp.sum(z * z, axis=1)[None, :]
    cn = jnp.sum(cb * cb, axis=1)[:, None]

    out_shapes = (
        jax.ShapeDtypeStruct((B, DIM), jnp.float32),
        jax.ShapeDtypeStruct((B, N_CODES), jnp.float32),
        jax.ShapeDtypeStruct((NB, 1, BB), jnp.int32),
        jax.ShapeDtypeStruct((1, 1), jnp.float32),
        jax.ShapeDtypeStruct((1, 1), jnp.float32),
        jax.ShapeDtypeStruct((1, 1), jnp.float32),
    )
    grid = (NB,)
    in_specs = [
        pl.BlockSpec((BB, DIM), lambda i: (i, 0)),
        pl.BlockSpec((N_CODES, DIM), lambda i: (0, 0)),
        pl.BlockSpec((N_CODES, DIM), lambda i: (0, 0)),
        pl.BlockSpec((1, BB), lambda i: (0, i)),
        pl.BlockSpec((N_CODES, 1), lambda i: (0, 0)),
    ]
    out_specs = (
        pl.BlockSpec((BB, DIM), lambda i: (i, 0)),
        pl.BlockSpec((BB, N_CODES), lambda i: (i, 0)),
        pl.BlockSpec((1, 1, BB), lambda i: (i, 0, 0)),
        pl.BlockSpec((1, 1), lambda i: (0, 0)),
        pl.BlockSpec((1, 1), lambda i: (0, 0)),
        pl.BlockSpec((1, 1), lambda i: (0, 0)),
    )
    zq, enc, idx3, closs, plex, usage = pl.pallas_call(
        _vq_kernel,
        grid=grid,
        in_specs=in_specs,
        out_specs=out_specs,
        out_shape=out_shapes,
        scratch_shapes=[
            pltpu.VMEM((1, N_CODES), jnp.float32),
            pltpu.SMEM((1, 1), jnp.float32),
        ],
    )(z, cb, cbm2, zn, cn)

    indices = idx3.reshape(B)
    return (zq.astype(z_e.dtype),
            closs.reshape(()),
            plex.reshape(()),
            usage.reshape(()),
            indices,
            enc.astype(z_e.dtype))
